# speedup vs baseline: 1.6737x; 1.6737x over previous
import jax
import jax.numpy as jnp
from jax.experimental import pallas as pl
from jax.experimental.pallas import tpu as pltpu

D_MODEL = 768
N_EXPERTS = 8
TOP_K = 2
D_FF = 768
T_TOKENS = 2048
T_CHUNK = 256


def _moe_dense_body(x_ref, gate_w_ref, gup_ref, down_ref, out_ref,
                    dw_ref, acc_ref, xbf_ref):
    e = pl.program_id(0)

    @pl.when(e == 0)
    def _router():
        x = x_ref[...]
        xbf_ref[...] = x.astype(jnp.bfloat16)
        logits = jnp.dot(x, gate_w_ref[...], preferred_element_type=jnp.float32)
        p = jax.nn.softmax(logits, axis=-1)
        lane = jax.lax.broadcasted_iota(jnp.int32, p.shape, 1)
        m1 = jnp.max(p, axis=1, keepdims=True)
        i1 = jnp.min(jnp.where(p == m1, lane, N_EXPERTS), axis=1, keepdims=True)
        sel1 = lane == i1
        p2 = jnp.where(sel1, -1.0, p)
        m2 = jnp.max(p2, axis=1, keepdims=True)
        i2 = jnp.min(jnp.where(p2 == m2, lane, N_EXPERTS), axis=1, keepdims=True)
        sel2 = lane == i2
        s = m1 + m2
        dw_ref[...] = jnp.where(sel1, m1 / s, 0.0) + jnp.where(sel2, m2 / s, 0.0)
        acc_ref[...] = jnp.zeros_like(acc_ref)

    gup = gup_ref[0].astype(jnp.bfloat16)
    down = down_ref[0].astype(jnp.bfloat16)
    lane = jax.lax.broadcasted_iota(jnp.int32, (T_CHUNK, N_EXPERTS), 1)

    def chunk(i, _):
        xs = xbf_ref[pl.ds(i * T_CHUNK, T_CHUNK), :]
        gu = jnp.dot(xs, gup, preferred_element_type=jnp.float32)
        g = gu[:, :D_FF]
        u = gu[:, D_FF:]
        act = (g * jax.nn.sigmoid(g) * u).astype(jnp.bfloat16)
        y = jnp.dot(act, down, preferred_element_type=jnp.float32)
        wfull = dw_ref[pl.ds(i * T_CHUNK, T_CHUNK), :]
        w = jnp.sum(jnp.where(lane == e, wfull, 0.0), axis=1, keepdims=True)
        acc_ref[pl.ds(i * T_CHUNK, T_CHUNK), :] += w * y
        return 0

    jax.lax.fori_loop(0, T_TOKENS // T_CHUNK, chunk, 0)

    @pl.when(e == N_EXPERTS - 1)
    def _flush():
        out_ref[...] = acc_ref[...]


def kernel(hidden_states, gate_w, gate_up_proj, down_proj):
    batch, seq, d = hidden_states.shape
    x = hidden_states.reshape(batch * seq, d)
    out = pl.pallas_call(
        _moe_dense_body,
        grid=(N_EXPERTS,),
        in_specs=[
            pl.BlockSpec((T_TOKENS, D_MODEL), lambda e: (0, 0)),
            pl.BlockSpec((D_MODEL, N_EXPERTS), lambda e: (0, 0)),
            pl.BlockSpec((1, D_MODEL, 2 * D_FF), lambda e: (e, 0, 0)),
            pl.BlockSpec((1, D_FF, D_MODEL), lambda e: (e, 0, 0)),
        ],
        out_specs=pl.BlockSpec((T_TOKENS, D_MODEL), lambda e: (0, 0)),
        out_shape=jax.ShapeDtypeStruct((T_TOKENS, D_MODEL), jnp.float32),
        scratch_shapes=[
            pltpu.VMEM((T_TOKENS, N_EXPERTS), jnp.float32),
            pltpu.VMEM((T_TOKENS, D_MODEL), jnp.float32),
            pltpu.VMEM((T_TOKENS, D_MODEL), jnp.bfloat16),
        ],
    )(x, gate_w, gate_up_proj, down_proj)
    return out.reshape(batch, seq, d)
